# baseline (speedup 1.0000x reference)
import functools

import jax
import jax.numpy as jnp
from jax import lax
from jax.experimental import pallas as pl
from jax.experimental.pallas import tpu as pltpu
from jax.experimental.pallas import tpu_sc as plsc

_N = 10000
_E = 320000
_NC = 2
_NS = 16
_B = 50
_D = 4
_RPT = 624
_TAIL = _N - _NS * _RPT
_BN = 400
_GRID = _N // _BN


@functools.cache
def _sc_segsum(n_chunks, dc, edge_split):
    wide = dc == 128
    bsz = _B
    ept = _E // (_NC * _NS) if edge_split else _E // _NS
    nb = ept // bsz
    g_blk = 40 if wide else nb
    nblk = nb // g_blk
    cpc = 1 if edge_split else n_chunks // _NC
    nlive = cpc * nblk
    out0 = _NC if edge_split else n_chunks
    mesh = plsc.VectorSubcoreMesh(core_axis_name="c", subcore_axis_name="s")

    def body(slab, e3, zrows, out, idx_s, idx_d, gbuf, acc, *sems):
        gsems, ssems, isem = sems[:_D], sems[_D:2 * _D], sems[2 * _D]
        k = lax.axis_index("c")
        s = lax.axis_index("s")
        row0 = ((k * _NS + s) if edge_split else s) * nb

        def blk_copies(bi, slot):
            rows = row0 + bi * g_blk
            return (pltpu.make_async_copy(e3.at[0].at[pl.ds(rows, g_blk)],
                                          idx_s.at[slot], isem),
                    pltpu.make_async_copy(e3.at[1].at[pl.ds(rows, g_blk)],
                                          idx_d.at[slot], isem))

        for d in blk_copies(0, 0):
            d.start()

        def mk_gather(chunk, slot, i, b):
            return pltpu.make_async_copy(
                chunk.at[idx_s.at[slot].at[i]], gbuf.at[b], gsems[b])

        def mk_scatter(slot, i, b):
            return pltpu.make_async_copy(
                gbuf.at[b], acc.at[idx_d.at[slot].at[i]], ssems[b])

        def chunk_ref(j):
            return slab.at[0 if edge_split else k * cpc + j]

        gi = 0
        for j in range(cpc):
            c = 0 if edge_split else k * cpc + j
            pltpu.sync_copy(zrows, acc.at[pl.ds(s * _RPT, _RPT)])

            @pl.when(s == _NS - 1)
            def _():
                pltpu.sync_copy(zrows.at[pl.ds(0, _TAIL)],
                                acc.at[pl.ds(_NS * _RPT, _TAIL)])

            if gi == 0:
                for d in blk_copies(0, 0):
                    d.wait()
                for b in range(_D - 1):
                    mk_gather(chunk_ref(0), 0, b, b).start()

            plsc.subcore_barrier()
            chunk = chunk_ref(j)

            for blk in range(nblk):
                slot = gi % 2
                nxt = gi + 1
                has_next = nxt < nlive
                if has_next:
                    for d in blk_copies((blk + 1) % nblk, nxt % 2):
                        d.start()

                def outer(gg, carry, chunk=chunk, slot=slot):
                    i0 = _D * gg
                    for b in range(_D):
                        i = i0 + b
                        mk_gather(chunk, slot, i, b).wait()
                        mk_scatter(slot, i, b).start(add=True)
                        nxtb = (b + _D - 1) % _D

                        @pl.when(i + _D - 1 < g_blk)
                        def _():
                            @pl.when(i >= 1)
                            def _():
                                mk_scatter(slot, i - 1, nxtb).wait()
                            mk_gather(chunk, slot, i + _D - 1, nxtb).start()

                    return carry

                lax.fori_loop(0, g_blk // _D, outer, 0)

                if has_next:
                    for d in blk_copies((blk + 1) % nblk, nxt % 2):
                        d.wait()
                    nchunk = chunk if blk + 1 < nblk else chunk_ref(j + 1)
                    nslot = nxt % 2
                    for b2 in range(_D - 1):
                        mk_scatter(slot, g_blk - _D + b2, b2).wait()
                        mk_gather(nchunk, nslot, b2, b2).start()
                    mk_scatter(slot, g_blk - 1, _D - 1).wait()
                else:
                    for b in range(_D):
                        mk_scatter(slot, g_blk - _D + b, b).wait()
                gi += 1

            plsc.subcore_barrier()
            oc = k if edge_split else c
            pltpu.sync_copy(acc.at[pl.ds(s * _RPT, _RPT)],
                            out.at[oc].at[pl.ds(s * _RPT, _RPT)])

            @pl.when(s == _NS - 1)
            def _():
                pltpu.sync_copy(acc.at[pl.ds(_NS * _RPT, _TAIL)],
                                out.at[oc].at[pl.ds(_NS * _RPT, _TAIL)])

    cparams = None if wide else pltpu.CompilerParams(use_tc_tiling_on_sc=False)
    return pl.kernel(
        body,
        out_type=jax.ShapeDtypeStruct((out0, _N, dc), jnp.float32),
        mesh=mesh,
        compiler_params=cparams,
        scratch_types=[
            pltpu.VMEM((2, g_blk, _B), jnp.int32),
            pltpu.VMEM((2, g_blk, _B), jnp.int32),
            pltpu.VMEM((_D, _B, dc), jnp.float32),
            pltpu.VMEM_SHARED((_N, dc), jnp.float32),
        ] + [pltpu.SemaphoreType.DMA] * (2 * _D + 1),
    )


def _row_spec(width):
    return pl.BlockSpec((_BN, width), lambda i: (i, 0))


def _slab_spec(c, width):
    return pl.BlockSpec((c, _BN, width), lambda i: (0, i, 0))


def _full_spec(a, b):
    return pl.BlockSpec((a, b), lambda i: (0, 0))


def _cat_norm(agg, sl, dv):
    t = agg[...] + sl[...]
    c = t.shape[0]
    return dv * jnp.concatenate([t[i] for i in range(c)], axis=-1)


def _store_slab(out, dv, h):
    c = out.shape[0]
    dc = out.shape[2]
    for i in range(c):
        out[i] = dv * h[:, dc * i:dc * (i + 1)]


def _prep0_body(degp, x, dinv, xs):
    d = degp[...]
    deg = d[0][:, 0:1] + d[1][:, 0:1] + 1.0
    dv = lax.rsqrt(deg)
    dinv[...] = dv
    xs[0] = dv * x[...]


def _prep0(degp, x):
    return pl.pallas_call(
        _prep0_body,
        grid=(_GRID,),
        in_specs=[_slab_spec(2, 16), _row_spec(128)],
        out_specs=[_row_spec(1), _slab_spec(1, 128)],
        out_shape=[jax.ShapeDtypeStruct((_N, 1), jnp.float32),
                   jax.ShapeDtypeStruct((1, _N, 128), jnp.float32)],
    )(degp, x)


def _l1_body(agg, xs, dinv, w, b, out):
    dv = dinv[...]
    a = agg[...]
    g = dv * (a[0] + a[1] + xs[0])
    h = jnp.maximum(jnp.dot(g, w[...], preferred_element_type=jnp.float32) + b[...], 0.0)
    _store_slab(out, dv, h)


def _l1(agg, xs, dinv, w, b):
    return pl.pallas_call(
        _l1_body,
        grid=(_GRID,),
        in_specs=[_slab_spec(2, 128), _slab_spec(1, 128), _row_spec(1),
                  _full_spec(128, 256), _full_spec(1, 256)],
        out_specs=_slab_spec(2, 128),
        out_shape=jax.ShapeDtypeStruct((2, _N, 128), jnp.float32),
    )(agg, xs, dinv, w, b)


def _l2_body(agg, sl, dinv, w, b, out):
    dv = dinv[...]
    g = _cat_norm(agg, sl, dv)
    h = jnp.maximum(jnp.dot(g, w[...], preferred_element_type=jnp.float32) + b[...], 0.0)
    _store_slab(out, dv, h)


def _l2(agg, sl, dinv, w, b):
    return pl.pallas_call(
        _l2_body,
        grid=(_GRID,),
        in_specs=[_slab_spec(2, 128), _slab_spec(2, 128), _row_spec(1),
                  _full_spec(256, 1024), _full_spec(1, 1024)],
        out_specs=_slab_spec(8, 128),
        out_shape=jax.ShapeDtypeStruct((8, _N, 128), jnp.float32),
    )(agg, sl, dinv, w, b)


def _l3_body(agg, sl, dinv, w3, b3, w4, out):
    dv = dinv[...]
    g = _cat_norm(agg, sl, dv)
    h3 = jnp.maximum(jnp.dot(g, w3[...], preferred_element_type=jnp.float32) + b3[...], 0.0)
    p4 = jnp.dot(h3, w4[...], preferred_element_type=jnp.float32)
    _store_slab(out, dv, p4)


def _l3(agg, sl, dinv, w3, b3, w4):
    return pl.pallas_call(
        _l3_body,
        grid=(_GRID,),
        in_specs=[_slab_spec(8, 128), _slab_spec(8, 128), _row_spec(1),
                  _full_spec(1024, 1024), _full_spec(1, 1024), _full_spec(1024, 256)],
        out_specs=_slab_spec(2, 128),
        out_shape=jax.ShapeDtypeStruct((2, _N, 128), jnp.float32),
    )(agg, sl, dinv, w3, b3, w4)


def _l4_body(agg, sl, dinv, b4, w5, out):
    dv = dinv[...]
    g = _cat_norm(agg, sl, dv)
    h4 = jnp.maximum(g + b4[...], 0.0)
    p5 = jnp.dot(h4, w5[...], preferred_element_type=jnp.float32)
    out[0] = dv * p5


def _l4(agg, sl, dinv, b4, w5):
    return pl.pallas_call(
        _l4_body,
        grid=(_GRID,),
        in_specs=[_slab_spec(2, 128), _slab_spec(2, 128), _row_spec(1),
                  _full_spec(1, 256), _full_spec(256, 16)],
        out_specs=_slab_spec(1, 16),
        out_shape=jax.ShapeDtypeStruct((1, _N, 16), jnp.float32),
    )(agg, sl, dinv, b4, w5)


def _l5_body(agg, sl, dinv, b5, out):
    dv = dinv[...]
    a = agg[...]
    v = dv * (a[0] + a[1] + sl[0])
    out[...] = v[:, :6] + b5[...]


def _l5(agg, sl, dinv, b5):
    return pl.pallas_call(
        _l5_body,
        grid=(_GRID,),
        in_specs=[_slab_spec(2, 16), _slab_spec(1, 16), _row_spec(1),
                  _full_spec(1, 6)],
        out_specs=_row_spec(6),
        out_shape=jax.ShapeDtypeStruct((_N, 6), jnp.float32),
    )(agg, sl, dinv, b5)


def kernel(x, edge_index, W1, b1, W2, b2, W3, b3, W4, b4, W5, b5):
    e3 = edge_index.reshape(2, _E // _B, _B)
    z128 = jnp.zeros((_RPT, 128), jnp.float32)
    z16 = jnp.zeros((_RPT, 16), jnp.float32)
    ones16 = jnp.ones((1, _N, 16), jnp.float32)

    degp = _sc_segsum(1, 16, True)(ones16, e3, z16)
    dinv, xslab = _prep0(degp, x)

    agg1 = _sc_segsum(1, 128, True)(xslab, e3, z128)
    slab2 = _l1(agg1, xslab, dinv, W1, b1.reshape(1, -1))

    agg2 = _sc_segsum(2, 128, False)(slab2, e3, z128)
    slab3 = _l2(agg2, slab2, dinv, W2, b2.reshape(1, -1))

    agg3 = _sc_segsum(8, 128, False)(slab3, e3, z128)
    slab4 = _l3(agg3, slab3, dinv, W3, b3.reshape(1, -1), W4)

    agg4 = _sc_segsum(2, 128, False)(slab4, e3, z128)
    w5p = jnp.zeros((256, 16), jnp.float32).at[:, :6].set(W5)
    slab5 = _l4(agg4, slab4, dinv, b4.reshape(1, -1), w5p)

    agg5 = _sc_segsum(1, 16, True)(slab5, e3, z16)
    return _l5(agg5, slab5, dinv, b5.reshape(1, -1))


# speedup vs baseline: 1.0322x; 1.0322x over previous
import functools

import jax
import jax.numpy as jnp
from jax import lax
from jax.experimental import pallas as pl
from jax.experimental.pallas import tpu as pltpu
from jax.experimental.pallas import tpu_sc as plsc

_N = 10000
_E = 320000
_NC = 2
_NS = 16
_B = 50
_BNAR = 125
_D = 4
_RPT = 624
_TAIL = _N - _NS * _RPT
_BN = 400
_GRID = _N // _BN


@functools.cache
def _sc_segsum(n_chunks, dc, edge_split):
    wide = dc == 128
    bsz = _B if wide else _BNAR
    ept = _E // (_NC * _NS) if edge_split else _E // _NS
    nb = ept // bsz
    g_blk = 40 if wide else nb
    nblk = nb // g_blk
    cpc = 1 if edge_split else n_chunks // _NC
    nlive = cpc * nblk
    out0 = _NC if edge_split else n_chunks
    mesh = plsc.VectorSubcoreMesh(core_axis_name="c", subcore_axis_name="s")

    def body(slab, src2, dst2, zrows, out, idx_s, idx_d, gbuf, acc, *sems):
        gsems, ssems, isem = sems[:_D], sems[_D:2 * _D], sems[2 * _D]
        k = lax.axis_index("c")
        s = lax.axis_index("s")
        row0 = ((k * _NS + s) if edge_split else s) * nb

        def blk_copies(bi, slot):
            rows = row0 + bi * g_blk
            return (pltpu.make_async_copy(src2.at[pl.ds(rows, g_blk)],
                                          idx_s.at[slot], isem),
                    pltpu.make_async_copy(dst2.at[pl.ds(rows, g_blk)],
                                          idx_d.at[slot], isem))

        for d in blk_copies(0, 0):
            d.start()

        def mk_gather(chunk, slot, i, b):
            return pltpu.make_async_copy(
                chunk.at[idx_s.at[slot].at[i]], gbuf.at[b], gsems[b])

        def mk_scatter(slot, i, b):
            return pltpu.make_async_copy(
                gbuf.at[b], acc.at[idx_d.at[slot].at[i]], ssems[b])

        def chunk_ref(j):
            return slab.at[0 if edge_split else k * cpc + j]

        gi = 0
        for j in range(cpc):
            c = 0 if edge_split else k * cpc + j
            pltpu.sync_copy(zrows, acc.at[pl.ds(s * _RPT, _RPT)])

            @pl.when(s == _NS - 1)
            def _():
                pltpu.sync_copy(zrows.at[pl.ds(0, _TAIL)],
                                acc.at[pl.ds(_NS * _RPT, _TAIL)])

            if gi == 0:
                for d in blk_copies(0, 0):
                    d.wait()
                for b in range(_D - 1):
                    mk_gather(chunk_ref(0), 0, b, b).start()

            plsc.subcore_barrier()
            chunk = chunk_ref(j)

            for blk in range(nblk):
                slot = gi % 2
                nxt = gi + 1
                has_next = nxt < nlive
                if has_next:
                    for d in blk_copies((blk + 1) % nblk, nxt % 2):
                        d.start()

                def outer(gg, carry, chunk=chunk, slot=slot):
                    i0 = _D * gg
                    for b in range(_D):
                        i = i0 + b
                        mk_gather(chunk, slot, i, b).wait()
                        mk_scatter(slot, i, b).start(add=True)
                        nxtb = (b + _D - 1) % _D

                        @pl.when(i + _D - 1 < g_blk)
                        def _():
                            @pl.when(i >= 1)
                            def _():
                                mk_scatter(slot, i - 1, nxtb).wait()
                            mk_gather(chunk, slot, i + _D - 1, nxtb).start()

                    return carry

                lax.fori_loop(0, g_blk // _D, outer, 0)

                if has_next:
                    for d in blk_copies((blk + 1) % nblk, nxt % 2):
                        d.wait()
                    nchunk = chunk if blk + 1 < nblk else chunk_ref(j + 1)
                    nslot = nxt % 2
                    for b2 in range(_D - 1):
                        mk_scatter(slot, g_blk - _D + b2, b2).wait()
                        mk_gather(nchunk, nslot, b2, b2).start()
                    mk_scatter(slot, g_blk - 1, _D - 1).wait()
                else:
                    for b in range(_D):
                        mk_scatter(slot, g_blk - _D + b, b).wait()
                gi += 1

            plsc.subcore_barrier()
            oc = k if edge_split else c
            pltpu.sync_copy(acc.at[pl.ds(s * _RPT, _RPT)],
                            out.at[oc].at[pl.ds(s * _RPT, _RPT)])

            @pl.when(s == _NS - 1)
            def _():
                pltpu.sync_copy(acc.at[pl.ds(_NS * _RPT, _TAIL)],
                                out.at[oc].at[pl.ds(_NS * _RPT, _TAIL)])

    cparams = None if wide else pltpu.CompilerParams(use_tc_tiling_on_sc=False)
    return pl.kernel(
        body,
        out_type=jax.ShapeDtypeStruct((out0, _N, dc), jnp.float32),
        mesh=mesh,
        compiler_params=cparams,
        scratch_types=[
            pltpu.VMEM((2, g_blk, bsz), jnp.int32),
            pltpu.VMEM((2, g_blk, bsz), jnp.int32),
            pltpu.VMEM((_D, bsz, dc), jnp.float32),
            pltpu.VMEM_SHARED((_N, dc), jnp.float32),
        ] + [pltpu.SemaphoreType.DMA] * (2 * _D + 1),
    )


def _row_spec(width):
    return pl.BlockSpec((_BN, width), lambda i: (i, 0))


def _slab_spec(c, width):
    return pl.BlockSpec((c, _BN, width), lambda i: (0, i, 0))


def _full_spec(a, b):
    return pl.BlockSpec((a, b), lambda i: (0, 0))


def _cat_norm(agg, sl, dv):
    t = agg[...] + sl[...]
    c = t.shape[0]
    return dv * jnp.concatenate([t[i] for i in range(c)], axis=-1)


def _store_slab(out, dv, h):
    c = out.shape[0]
    dc = out.shape[2]
    for i in range(c):
        out[i] = dv * h[:, dc * i:dc * (i + 1)]


def _prep0_body(degp, x, dinv, xs):
    d = degp[...]
    deg = d[0][:, 0:1] + d[1][:, 0:1] + 1.0
    dv = lax.rsqrt(deg)
    dinv[...] = dv
    xs[0] = dv * x[...]


def _prep0(degp, x):
    return pl.pallas_call(
        _prep0_body,
        grid=(_GRID,),
        in_specs=[_slab_spec(2, 16), _row_spec(128)],
        out_specs=[_row_spec(1), _slab_spec(1, 128)],
        out_shape=[jax.ShapeDtypeStruct((_N, 1), jnp.float32),
                   jax.ShapeDtypeStruct((1, _N, 128), jnp.float32)],
    )(degp, x)


def _l1_body(agg, xs, dinv, w, b, out):
    dv = dinv[...]
    a = agg[...]
    g = dv * (a[0] + a[1] + xs[0])
    h = jnp.maximum(jnp.dot(g, w[...], preferred_element_type=jnp.float32) + b[...], 0.0)
    _store_slab(out, dv, h)


def _l1(agg, xs, dinv, w, b):
    return pl.pallas_call(
        _l1_body,
        grid=(_GRID,),
        in_specs=[_slab_spec(2, 128), _slab_spec(1, 128), _row_spec(1),
                  _full_spec(128, 256), _full_spec(1, 256)],
        out_specs=_slab_spec(2, 128),
        out_shape=jax.ShapeDtypeStruct((2, _N, 128), jnp.float32),
    )(agg, xs, dinv, w, b)


def _l2_body(agg, sl, dinv, w, b, out):
    dv = dinv[...]
    g = _cat_norm(agg, sl, dv)
    h = jnp.maximum(jnp.dot(g, w[...], preferred_element_type=jnp.float32) + b[...], 0.0)
    _store_slab(out, dv, h)


def _l2(agg, sl, dinv, w, b):
    return pl.pallas_call(
        _l2_body,
        grid=(_GRID,),
        in_specs=[_slab_spec(2, 128), _slab_spec(2, 128), _row_spec(1),
                  _full_spec(256, 1024), _full_spec(1, 1024)],
        out_specs=_slab_spec(8, 128),
        out_shape=jax.ShapeDtypeStruct((8, _N, 128), jnp.float32),
    )(agg, sl, dinv, w, b)


def _l3_body(agg, sl, dinv, w3, b3, w4, out):
    dv = dinv[...]
    g = _cat_norm(agg, sl, dv)
    h3 = jnp.maximum(jnp.dot(g, w3[...], preferred_element_type=jnp.float32) + b3[...], 0.0)
    p4 = jnp.dot(h3, w4[...], preferred_element_type=jnp.float32)
    _store_slab(out, dv, p4)


def _l3(agg, sl, dinv, w3, b3, w4):
    return pl.pallas_call(
        _l3_body,
        grid=(_GRID,),
        in_specs=[_slab_spec(8, 128), _slab_spec(8, 128), _row_spec(1),
                  _full_spec(1024, 1024), _full_spec(1, 1024), _full_spec(1024, 256)],
        out_specs=_slab_spec(2, 128),
        out_shape=jax.ShapeDtypeStruct((2, _N, 128), jnp.float32),
    )(agg, sl, dinv, w3, b3, w4)


def _l4_body(agg, sl, dinv, b4, w5, out):
    dv = dinv[...]
    g = _cat_norm(agg, sl, dv)
    h4 = jnp.maximum(g + b4[...], 0.0)
    p5 = jnp.dot(h4, w5[...], preferred_element_type=jnp.float32)
    out[0] = dv * p5


def _l4(agg, sl, dinv, b4, w5):
    return pl.pallas_call(
        _l4_body,
        grid=(_GRID,),
        in_specs=[_slab_spec(2, 128), _slab_spec(2, 128), _row_spec(1),
                  _full_spec(1, 256), _full_spec(256, 16)],
        out_specs=_slab_spec(1, 16),
        out_shape=jax.ShapeDtypeStruct((1, _N, 16), jnp.float32),
    )(agg, sl, dinv, b4, w5)


def _l5_body(agg, sl, dinv, b5, out):
    dv = dinv[...]
    a = agg[...]
    v = dv * (a[0] + a[1] + sl[0])
    out[...] = v[:, :6] + b5[...]


def _l5(agg, sl, dinv, b5):
    return pl.pallas_call(
        _l5_body,
        grid=(_GRID,),
        in_specs=[_slab_spec(2, 16), _slab_spec(1, 16), _row_spec(1),
                  _full_spec(1, 6)],
        out_specs=_row_spec(6),
        out_shape=jax.ShapeDtypeStruct((_N, 6), jnp.float32),
    )(agg, sl, dinv, b5)


def kernel(x, edge_index, W1, b1, W2, b2, W3, b3, W4, b4, W5, b5):
    srcw = edge_index[0].reshape(_E // _B, _B)
    dstw = edge_index[1].reshape(_E // _B, _B)
    srcn = edge_index[0].reshape(_E // _BNAR, _BNAR)
    dstn = edge_index[1].reshape(_E // _BNAR, _BNAR)
    z128 = jnp.zeros((_RPT, 128), jnp.float32)
    z16 = jnp.zeros((_RPT, 16), jnp.float32)
    ones16 = jnp.ones((1, _N, 16), jnp.float32)

    degp = _sc_segsum(1, 16, True)(ones16, srcn, dstn, z16)
    dinv, xslab = _prep0(degp, x)

    agg1 = _sc_segsum(1, 128, True)(xslab, srcw, dstw, z128)
    slab2 = _l1(agg1, xslab, dinv, W1, b1.reshape(1, -1))

    agg2 = _sc_segsum(2, 128, False)(slab2, srcw, dstw, z128)
    slab3 = _l2(agg2, slab2, dinv, W2, b2.reshape(1, -1))

    agg3 = _sc_segsum(8, 128, False)(slab3, srcw, dstw, z128)
    slab4 = _l3(agg3, slab3, dinv, W3, b3.reshape(1, -1), W4)

    agg4 = _sc_segsum(2, 128, False)(slab4, srcw, dstw, z128)
    w5p = jnp.zeros((256, 16), jnp.float32).at[:, :6].set(W5)
    slab5 = _l4(agg4, slab4, dinv, b4.reshape(1, -1), w5p)

    agg5 = _sc_segsum(1, 16, True)(slab5, srcn, dstn, z16)
    return _l5(agg5, slab5, dinv, b5.reshape(1, -1))


# speedup vs baseline: 1.0658x; 1.0325x over previous
import functools

import jax
import jax.numpy as jnp
from jax import lax
from jax.experimental import pallas as pl
from jax.experimental.pallas import tpu as pltpu
from jax.experimental.pallas import tpu_sc as plsc

_N = 10000
_E = 320000
_NC = 2
_NS = 16
_B = 50
_BNAR = 125
_D = 4
_RPT = 624
_TAIL = _N - _NS * _RPT
_BN = 1000
_GRID = _N // _BN


@functools.cache
def _sc_segsum(n_chunks, dc, edge_split):
    wide = dc == 128
    bsz = _B if wide else _BNAR
    ept = _E // (_NC * _NS) if edge_split else _E // _NS
    nb = ept // bsz
    g_blk = 40 if wide else nb
    nblk = nb // g_blk
    cpc = 1 if edge_split else n_chunks // _NC
    nlive = cpc * nblk
    out0 = _NC if edge_split else n_chunks
    mesh = plsc.VectorSubcoreMesh(core_axis_name="c", subcore_axis_name="s")

    def body(slab, src2, dst2, zrows, out, idx_s, idx_d, gbuf, acc, *sems):
        gsems, ssems, isem = sems[:_D], sems[_D:2 * _D], sems[2 * _D]
        k = lax.axis_index("c")
        s = lax.axis_index("s")
        row0 = ((k * _NS + s) if edge_split else s) * nb

        def blk_copies(bi, slot):
            rows = row0 + bi * g_blk
            return (pltpu.make_async_copy(src2.at[pl.ds(rows, g_blk)],
                                          idx_s.at[slot], isem),
                    pltpu.make_async_copy(dst2.at[pl.ds(rows, g_blk)],
                                          idx_d.at[slot], isem))

        for d in blk_copies(0, 0):
            d.start()

        def mk_gather(chunk, slot, i, b):
            return pltpu.make_async_copy(
                chunk.at[idx_s.at[slot].at[i]], gbuf.at[b], gsems[b])

        def mk_scatter(slot, i, b):
            return pltpu.make_async_copy(
                gbuf.at[b], acc.at[idx_d.at[slot].at[i]], ssems[b])

        def chunk_ref(j):
            return slab.at[0 if edge_split else k * cpc + j]

        gi = 0
        for j in range(cpc):
            c = 0 if edge_split else k * cpc + j
            pltpu.sync_copy(zrows, acc.at[pl.ds(s * _RPT, _RPT)])

            @pl.when(s == _NS - 1)
            def _():
                pltpu.sync_copy(zrows.at[pl.ds(0, _TAIL)],
                                acc.at[pl.ds(_NS * _RPT, _TAIL)])

            if gi == 0:
                for d in blk_copies(0, 0):
                    d.wait()
                for b in range(_D - 1):
                    mk_gather(chunk_ref(0), 0, b, b).start()

            plsc.subcore_barrier()
            chunk = chunk_ref(j)

            for blk in range(nblk):
                slot = gi % 2
                nxt = gi + 1
                has_next = nxt < nlive
                if has_next:
                    for d in blk_copies((blk + 1) % nblk, nxt % 2):
                        d.start()

                def outer(gg, carry, chunk=chunk, slot=slot):
                    i0 = _D * gg
                    for b in range(_D):
                        i = i0 + b
                        mk_gather(chunk, slot, i, b).wait()
                        mk_scatter(slot, i, b).start(add=True)
                        nxtb = (b + _D - 1) % _D

                        @pl.when(i + _D - 1 < g_blk)
                        def _():
                            @pl.when(i >= 1)
                            def _():
                                mk_scatter(slot, i - 1, nxtb).wait()
                            mk_gather(chunk, slot, i + _D - 1, nxtb).start()

                    return carry

                lax.fori_loop(0, g_blk // _D, outer, 0)

                if has_next:
                    for d in blk_copies((blk + 1) % nblk, nxt % 2):
                        d.wait()
                    nchunk = chunk if blk + 1 < nblk else chunk_ref(j + 1)
                    nslot = nxt % 2
                    for b2 in range(_D - 1):
                        mk_scatter(slot, g_blk - _D + b2, b2).wait()
                        mk_gather(nchunk, nslot, b2, b2).start()
                    mk_scatter(slot, g_blk - 1, _D - 1).wait()
                else:
                    for b in range(_D):
                        mk_scatter(slot, g_blk - _D + b, b).wait()
                gi += 1

            plsc.subcore_barrier()
            oc = k if edge_split else c
            pltpu.sync_copy(acc.at[pl.ds(s * _RPT, _RPT)],
                            out.at[oc].at[pl.ds(s * _RPT, _RPT)])

            @pl.when(s == _NS - 1)
            def _():
                pltpu.sync_copy(acc.at[pl.ds(_NS * _RPT, _TAIL)],
                                out.at[oc].at[pl.ds(_NS * _RPT, _TAIL)])

    cparams = None if wide else pltpu.CompilerParams(use_tc_tiling_on_sc=False)
    return pl.kernel(
        body,
        out_type=jax.ShapeDtypeStruct((out0, _N, dc), jnp.float32),
        mesh=mesh,
        compiler_params=cparams,
        scratch_types=[
            pltpu.VMEM((2, g_blk, bsz), jnp.int32),
            pltpu.VMEM((2, g_blk, bsz), jnp.int32),
            pltpu.VMEM((_D, bsz, dc), jnp.float32),
            pltpu.VMEM_SHARED((_N, dc), jnp.float32),
        ] + [pltpu.SemaphoreType.DMA] * (2 * _D + 1),
    )


def _row_spec(width):
    return pl.BlockSpec((_BN, width), lambda i: (i, 0))


def _slab_spec(c, width):
    return pl.BlockSpec((c, _BN, width), lambda i: (0, i, 0))


def _full_spec(a, b):
    return pl.BlockSpec((a, b), lambda i: (0, 0))


def _cat_norm(agg, sl, dv):
    t = agg[...] + sl[...]
    c = t.shape[0]
    return dv * jnp.concatenate([t[i] for i in range(c)], axis=-1)


def _store_slab(out, dv, h):
    c = out.shape[0]
    dc = out.shape[2]
    for i in range(c):
        out[i] = dv * h[:, dc * i:dc * (i + 1)]


def _prep0_body(degp, x, dinv, xs):
    d = degp[...]
    deg = d[0][:, 0:1] + d[1][:, 0:1] + 1.0
    dv = lax.rsqrt(deg)
    dinv[...] = dv
    xs[0] = dv * x[...]


def _prep0(degp, x):
    return pl.pallas_call(
        _prep0_body,
        grid=(_GRID,),
        in_specs=[_slab_spec(2, 16), _row_spec(128)],
        out_specs=[_row_spec(1), _slab_spec(1, 128)],
        out_shape=[jax.ShapeDtypeStruct((_N, 1), jnp.float32),
                   jax.ShapeDtypeStruct((1, _N, 128), jnp.float32)],
    )(degp, x)


def _l1_body(agg, xs, dinv, w, b, out):
    dv = dinv[...]
    a = agg[...]
    g = dv * (a[0] + a[1] + xs[0])
    h = jnp.maximum(jnp.dot(g, w[...], preferred_element_type=jnp.float32) + b[...], 0.0)
    _store_slab(out, dv, h)


def _l1(agg, xs, dinv, w, b):
    return pl.pallas_call(
        _l1_body,
        grid=(_GRID,),
        in_specs=[_slab_spec(2, 128), _slab_spec(1, 128), _row_spec(1),
                  _full_spec(128, 256), _full_spec(1, 256)],
        out_specs=_slab_spec(2, 128),
        out_shape=jax.ShapeDtypeStruct((2, _N, 128), jnp.float32),
    )(agg, xs, dinv, w, b)


def _l2_body(agg, sl, dinv, w, b, out):
    dv = dinv[...]
    g = _cat_norm(agg, sl, dv)
    h = jnp.maximum(jnp.dot(g, w[...], preferred_element_type=jnp.float32) + b[...], 0.0)
    _store_slab(out, dv, h)


def _l2(agg, sl, dinv, w, b):
    return pl.pallas_call(
        _l2_body,
        grid=(_GRID,),
        in_specs=[_slab_spec(2, 128), _slab_spec(2, 128), _row_spec(1),
                  _full_spec(256, 1024), _full_spec(1, 1024)],
        out_specs=_slab_spec(8, 128),
        out_shape=jax.ShapeDtypeStruct((8, _N, 128), jnp.float32),
    )(agg, sl, dinv, w, b)


def _l3_body(agg, sl, dinv, w3, b3, w4, out):
    dv = dinv[...]
    g = _cat_norm(agg, sl, dv)
    h3 = jnp.maximum(jnp.dot(g, w3[...], preferred_element_type=jnp.float32) + b3[...], 0.0)
    p4 = jnp.dot(h3, w4[...], preferred_element_type=jnp.float32)
    _store_slab(out, dv, p4)


def _l3(agg, sl, dinv, w3, b3, w4):
    return pl.pallas_call(
        _l3_body,
        grid=(_GRID,),
        in_specs=[_slab_spec(8, 128), _slab_spec(8, 128), _row_spec(1),
                  _full_spec(1024, 1024), _full_spec(1, 1024), _full_spec(1024, 256)],
        out_specs=_slab_spec(2, 128),
        out_shape=jax.ShapeDtypeStruct((2, _N, 128), jnp.float32),
    )(agg, sl, dinv, w3, b3, w4)


def _l4_body(agg, sl, dinv, b4, w5, out):
    dv = dinv[...]
    g = _cat_norm(agg, sl, dv)
    h4 = jnp.maximum(g + b4[...], 0.0)
    p5 = jnp.dot(h4, w5[...], preferred_element_type=jnp.float32)
    out[0] = dv * p5


def _l4(agg, sl, dinv, b4, w5):
    return pl.pallas_call(
        _l4_body,
        grid=(_GRID,),
        in_specs=[_slab_spec(2, 128), _slab_spec(2, 128), _row_spec(1),
                  _full_spec(1, 256), _full_spec(256, 16)],
        out_specs=_slab_spec(1, 16),
        out_shape=jax.ShapeDtypeStruct((1, _N, 16), jnp.float32),
    )(agg, sl, dinv, b4, w5)


def _l5_body(agg, sl, dinv, b5, out):
    dv = dinv[...]
    a = agg[...]
    v = dv * (a[0] + a[1] + sl[0])
    out[...] = v[:, :6] + b5[...]


def _l5(agg, sl, dinv, b5):
    return pl.pallas_call(
        _l5_body,
        grid=(_GRID,),
        in_specs=[_slab_spec(2, 16), _slab_spec(1, 16), _row_spec(1),
                  _full_spec(1, 6)],
        out_specs=_row_spec(6),
        out_shape=jax.ShapeDtypeStruct((_N, 6), jnp.float32),
    )(agg, sl, dinv, b5)


def kernel(x, edge_index, W1, b1, W2, b2, W3, b3, W4, b4, W5, b5):
    srcw = edge_index[0].reshape(_E // _B, _B)
    dstw = edge_index[1].reshape(_E // _B, _B)
    srcn = edge_index[0].reshape(_E // _BNAR, _BNAR)
    dstn = edge_index[1].reshape(_E // _BNAR, _BNAR)
    z128 = jnp.zeros((_RPT, 128), jnp.float32)
    z16 = jnp.zeros((_RPT, 16), jnp.float32)
    ones16 = jnp.ones((1, _N, 16), jnp.float32)

    degp = _sc_segsum(1, 16, True)(ones16, srcn, dstn, z16)
    dinv, xslab = _prep0(degp, x)

    agg1 = _sc_segsum(1, 128, True)(xslab, srcw, dstw, z128)
    slab2 = _l1(agg1, xslab, dinv, W1, b1.reshape(1, -1))

    agg2 = _sc_segsum(2, 128, False)(slab2, srcw, dstw, z128)
    slab3 = _l2(agg2, slab2, dinv, W2, b2.reshape(1, -1))

    agg3 = _sc_segsum(8, 128, False)(slab3, srcw, dstw, z128)
    slab4 = _l3(agg3, slab3, dinv, W3, b3.reshape(1, -1), W4)

    agg4 = _sc_segsum(2, 128, False)(slab4, srcw, dstw, z128)
    w5p = jnp.zeros((256, 16), jnp.float32).at[:, :6].set(W5)
    slab5 = _l4(agg4, slab4, dinv, b4.reshape(1, -1), w5p)

    agg5 = _sc_segsum(1, 16, True)(slab5, srcn, dstn, z16)
    return _l5(agg5, slab5, dinv, b5.reshape(1, -1))


# speedup vs baseline: 1.0709x; 1.0048x over previous
import functools

import jax
import jax.numpy as jnp
from jax import lax
from jax.experimental import pallas as pl
from jax.experimental.pallas import tpu as pltpu
from jax.experimental.pallas import tpu_sc as plsc

_N = 10000
_E = 320000
_NC = 2
_NS = 16
_B = 50
_BNAR = 125
_D = 4
_RPT = 624
_TAIL = _N - _NS * _RPT
_BN = 2000
_GRID = _N // _BN


@functools.cache
def _sc_segsum(n_chunks, dc, edge_split):
    wide = dc == 128
    bsz = _B if wide else _BNAR
    ept = _E // (_NC * _NS) if edge_split else _E // _NS
    nb = ept // bsz
    g_blk = 40 if wide else nb
    nblk = nb // g_blk
    cpc = 1 if edge_split else n_chunks // _NC
    nlive = cpc * nblk
    out0 = _NC if edge_split else n_chunks
    mesh = plsc.VectorSubcoreMesh(core_axis_name="c", subcore_axis_name="s")

    def body(slab, src2, dst2, zrows, out, idx_s, idx_d, gbuf, acc, *sems):
        gsems, ssems, isem = sems[:_D], sems[_D:2 * _D], sems[2 * _D]
        k = lax.axis_index("c")
        s = lax.axis_index("s")
        row0 = ((k * _NS + s) if edge_split else s) * nb

        def blk_copies(bi, slot):
            rows = row0 + bi * g_blk
            return (pltpu.make_async_copy(src2.at[pl.ds(rows, g_blk)],
                                          idx_s.at[slot], isem),
                    pltpu.make_async_copy(dst2.at[pl.ds(rows, g_blk)],
                                          idx_d.at[slot], isem))

        for d in blk_copies(0, 0):
            d.start()

        def mk_gather(chunk, slot, i, b):
            return pltpu.make_async_copy(
                chunk.at[idx_s.at[slot].at[i]], gbuf.at[b], gsems[b])

        def mk_scatter(slot, i, b):
            return pltpu.make_async_copy(
                gbuf.at[b], acc.at[idx_d.at[slot].at[i]], ssems[b])

        def chunk_ref(j):
            return slab.at[0 if edge_split else k * cpc + j]

        gi = 0
        for j in range(cpc):
            c = 0 if edge_split else k * cpc + j
            pltpu.sync_copy(zrows, acc.at[pl.ds(s * _RPT, _RPT)])

            @pl.when(s == _NS - 1)
            def _():
                pltpu.sync_copy(zrows.at[pl.ds(0, _TAIL)],
                                acc.at[pl.ds(_NS * _RPT, _TAIL)])

            if gi == 0:
                for d in blk_copies(0, 0):
                    d.wait()
                for b in range(_D - 1):
                    mk_gather(chunk_ref(0), 0, b, b).start()

            plsc.subcore_barrier()
            chunk = chunk_ref(j)

            for blk in range(nblk):
                slot = gi % 2
                nxt = gi + 1
                has_next = nxt < nlive
                if has_next:
                    for d in blk_copies((blk + 1) % nblk, nxt % 2):
                        d.start()

                def outer(gg, carry, chunk=chunk, slot=slot):
                    i0 = _D * gg
                    for b in range(_D):
                        i = i0 + b
                        mk_gather(chunk, slot, i, b).wait()
                        mk_scatter(slot, i, b).start(add=True)
                        nxtb = (b + _D - 1) % _D

                        @pl.when(i + _D - 1 < g_blk)
                        def _():
                            @pl.when(i >= 1)
                            def _():
                                mk_scatter(slot, i - 1, nxtb).wait()
                            mk_gather(chunk, slot, i + _D - 1, nxtb).start()

                    return carry

                lax.fori_loop(0, g_blk // _D, outer, 0)

                if has_next:
                    for d in blk_copies((blk + 1) % nblk, nxt % 2):
                        d.wait()
                    nchunk = chunk if blk + 1 < nblk else chunk_ref(j + 1)
                    nslot = nxt % 2
                    for b2 in range(_D - 1):
                        mk_scatter(slot, g_blk - _D + b2, b2).wait()
                        mk_gather(nchunk, nslot, b2, b2).start()
                    mk_scatter(slot, g_blk - 1, _D - 1).wait()
                else:
                    for b in range(_D):
                        mk_scatter(slot, g_blk - _D + b, b).wait()
                gi += 1

            plsc.subcore_barrier()
            oc = k if edge_split else c
            pltpu.sync_copy(acc.at[pl.ds(s * _RPT, _RPT)],
                            out.at[oc].at[pl.ds(s * _RPT, _RPT)])

            @pl.when(s == _NS - 1)
            def _():
                pltpu.sync_copy(acc.at[pl.ds(_NS * _RPT, _TAIL)],
                                out.at[oc].at[pl.ds(_NS * _RPT, _TAIL)])

    cparams = None if wide else pltpu.CompilerParams(use_tc_tiling_on_sc=False)
    return pl.kernel(
        body,
        out_type=jax.ShapeDtypeStruct((out0, _N, dc), jnp.float32),
        mesh=mesh,
        compiler_params=cparams,
        scratch_types=[
            pltpu.VMEM((2, g_blk, bsz), jnp.int32),
            pltpu.VMEM((2, g_blk, bsz), jnp.int32),
            pltpu.VMEM((_D, bsz, dc), jnp.float32),
            pltpu.VMEM_SHARED((_N, dc), jnp.float32),
        ] + [pltpu.SemaphoreType.DMA] * (2 * _D + 1),
    )


def _row_spec(width):
    return pl.BlockSpec((_BN, width), lambda i: (i, 0))


def _slab_spec(c, width):
    return pl.BlockSpec((c, _BN, width), lambda i: (0, i, 0))


def _full_spec(a, b):
    return pl.BlockSpec((a, b), lambda i: (0, 0))


def _cat_norm(agg, sl, dv):
    t = agg[...] + sl[...]
    c = t.shape[0]
    return dv * jnp.concatenate([t[i] for i in range(c)], axis=-1)


def _store_slab(out, dv, h):
    c = out.shape[0]
    dc = out.shape[2]
    for i in range(c):
        out[i] = dv * h[:, dc * i:dc * (i + 1)]


def _prep0_body(degp, x, dinv, xs):
    d = degp[...]
    deg = d[0][:, 0:1] + d[1][:, 0:1] + 1.0
    dv = lax.rsqrt(deg)
    dinv[...] = dv
    xs[0] = dv * x[...]


def _prep0(degp, x):
    return pl.pallas_call(
        _prep0_body,
        grid=(_GRID,),
        in_specs=[_slab_spec(2, 16), _row_spec(128)],
        out_specs=[_row_spec(1), _slab_spec(1, 128)],
        out_shape=[jax.ShapeDtypeStruct((_N, 1), jnp.float32),
                   jax.ShapeDtypeStruct((1, _N, 128), jnp.float32)],
    )(degp, x)


def _l1_body(agg, xs, dinv, w, b, out):
    dv = dinv[...]
    a = agg[...]
    g = dv * (a[0] + a[1] + xs[0])
    h = jnp.maximum(jnp.dot(g, w[...], preferred_element_type=jnp.float32) + b[...], 0.0)
    _store_slab(out, dv, h)


def _l1(agg, xs, dinv, w, b):
    return pl.pallas_call(
        _l1_body,
        grid=(_GRID,),
        in_specs=[_slab_spec(2, 128), _slab_spec(1, 128), _row_spec(1),
                  _full_spec(128, 256), _full_spec(1, 256)],
        out_specs=_slab_spec(2, 128),
        out_shape=jax.ShapeDtypeStruct((2, _N, 128), jnp.float32),
    )(agg, xs, dinv, w, b)


def _l2_body(agg, sl, dinv, w, b, out):
    dv = dinv[...]
    g = _cat_norm(agg, sl, dv)
    h = jnp.maximum(jnp.dot(g, w[...], preferred_element_type=jnp.float32) + b[...], 0.0)
    _store_slab(out, dv, h)


def _l2(agg, sl, dinv, w, b):
    return pl.pallas_call(
        _l2_body,
        grid=(_GRID,),
        in_specs=[_slab_spec(2, 128), _slab_spec(2, 128), _row_spec(1),
                  _full_spec(256, 1024), _full_spec(1, 1024)],
        out_specs=_slab_spec(8, 128),
        out_shape=jax.ShapeDtypeStruct((8, _N, 128), jnp.float32),
    )(agg, sl, dinv, w, b)


def _l3_body(agg, sl, dinv, w3, b3, w4, out):
    dv = dinv[...]
    g = _cat_norm(agg, sl, dv)
    h3 = jnp.maximum(jnp.dot(g, w3[...], preferred_element_type=jnp.float32) + b3[...], 0.0)
    p4 = jnp.dot(h3, w4[...], preferred_element_type=jnp.float32)
    _store_slab(out, dv, p4)


def _l3(agg, sl, dinv, w3, b3, w4):
    return pl.pallas_call(
        _l3_body,
        grid=(_GRID,),
        in_specs=[_slab_spec(8, 128), _slab_spec(8, 128), _row_spec(1),
                  _full_spec(1024, 1024), _full_spec(1, 1024), _full_spec(1024, 256)],
        out_specs=_slab_spec(2, 128),
        out_shape=jax.ShapeDtypeStruct((2, _N, 128), jnp.float32),
    )(agg, sl, dinv, w3, b3, w4)


def _l4_body(agg, sl, dinv, b4, w5, out):
    dv = dinv[...]
    g = _cat_norm(agg, sl, dv)
    h4 = jnp.maximum(g + b4[...], 0.0)
    p5 = jnp.dot(h4, w5[...], preferred_element_type=jnp.float32)
    out[0] = dv * p5


def _l4(agg, sl, dinv, b4, w5):
    return pl.pallas_call(
        _l4_body,
        grid=(_GRID,),
        in_specs=[_slab_spec(2, 128), _slab_spec(2, 128), _row_spec(1),
                  _full_spec(1, 256), _full_spec(256, 16)],
        out_specs=_slab_spec(1, 16),
        out_shape=jax.ShapeDtypeStruct((1, _N, 16), jnp.float32),
    )(agg, sl, dinv, b4, w5)


def _l5_body(agg, sl, dinv, b5, out):
    dv = dinv[...]
    a = agg[...]
    v = dv * (a[0] + a[1] + sl[0])
    out[...] = v[:, :6] + b5[...]


def _l5(agg, sl, dinv, b5):
    return pl.pallas_call(
        _l5_body,
        grid=(_GRID,),
        in_specs=[_slab_spec(2, 16), _slab_spec(1, 16), _row_spec(1),
                  _full_spec(1, 6)],
        out_specs=_row_spec(6),
        out_shape=jax.ShapeDtypeStruct((_N, 6), jnp.float32),
    )(agg, sl, dinv, b5)


def kernel(x, edge_index, W1, b1, W2, b2, W3, b3, W4, b4, W5, b5):
    srcw = edge_index[0].reshape(_E // _B, _B)
    dstw = edge_index[1].reshape(_E // _B, _B)
    srcn = edge_index[0].reshape(_E // _BNAR, _BNAR)
    dstn = edge_index[1].reshape(_E // _BNAR, _BNAR)
    z128 = jnp.zeros((_RPT, 128), jnp.float32)
    z16 = jnp.zeros((_RPT, 16), jnp.float32)
    ones16 = jnp.ones((1, _N, 16), jnp.float32)

    degp = _sc_segsum(1, 16, True)(ones16, srcn, dstn, z16)
    dinv, xslab = _prep0(degp, x)

    agg1 = _sc_segsum(1, 128, True)(xslab, srcw, dstw, z128)
    slab2 = _l1(agg1, xslab, dinv, W1, b1.reshape(1, -1))

    agg2 = _sc_segsum(2, 128, False)(slab2, srcw, dstw, z128)
    slab3 = _l2(agg2, slab2, dinv, W2, b2.reshape(1, -1))

    agg3 = _sc_segsum(8, 128, False)(slab3, srcw, dstw, z128)
    slab4 = _l3(agg3, slab3, dinv, W3, b3.reshape(1, -1), W4)

    agg4 = _sc_segsum(2, 128, False)(slab4, srcw, dstw, z128)
    w5p = jnp.zeros((256, 16), jnp.float32).at[:, :6].set(W5)
    slab5 = _l4(agg4, slab4, dinv, b4.reshape(1, -1), w5p)

    agg5 = _sc_segsum(1, 16, True)(slab5, srcn, dstn, z16)
    return _l5(agg5, slab5, dinv, b5.reshape(1, -1))
